# baseline (speedup 1.0000x reference)
import functools

import jax
import jax.numpy as jnp
from jax import lax
from jax.experimental import pallas as pl
from jax.experimental.pallas import tpu as pltpu
from jax.experimental.pallas import tpu_sc as plsc

VOCAB_N = 1000000
DIM_N = 100
HIDDEN_N = 150
CLASSES_N = 1000
BATCH_N = 4096
SEQ_N = 200

_NC = 2
_NS = 16
_NW = _NC * _NS
_BPW = BATCH_N // _NW
_OFFS = (0, 16, 32, 48, 64, 80, 84)
_LL = 128
_U0 = 112
_O1 = 88
_U1 = SEQ_N - _U0


_DP = 104


def _sc_mean_pool(idx_flat, table):
  mesh = plsc.VectorSubcoreMesh(core_axis_name="c", subcore_axis_name="s")

  @functools.partial(
      pl.kernel,
      mesh=mesh,
      out_type=jax.ShapeDtypeStruct((BATCH_N, DIM_N), jnp.float32),
      compiler_params=pltpu.CompilerParams(use_tc_tiling_on_sc=False),
      scratch_types=[
          pltpu.VMEM((2, _LL), jnp.int32),
          pltpu.VMEM((2, _LL, _DP), jnp.float32),
          pltpu.VMEM((_BPW, DIM_N), jnp.float32),
          pltpu.SemaphoreType.DMA,
          pltpu.SemaphoreType.DMA,
      ],
  )
  def k(idx_hbm, table_hbm, out_hbm, idx_v, rows_v, pooled_v, sem0, sem1):
    wid = lax.axis_index("s") * _NC + lax.axis_index("c")
    base = wid * _BPW
    scale = jnp.float32(1.0 / SEQ_N)

    def elt_body(e, carry):
      fb = (base + e) * SEQ_N
      pltpu.sync_copy(idx_hbm.at[pl.ds(fb, _LL)], idx_v.at[0])
      pltpu.sync_copy(idx_hbm.at[pl.ds(fb + _O1, _LL)], idx_v.at[1])
      cp0 = pltpu.async_copy(table_hbm.at[idx_v.at[0]], rows_v.at[0], sem0)
      cp1 = pltpu.async_copy(table_hbm.at[idx_v.at[1]], rows_v.at[1], sem1)
      cp0.wait()
      cp1.wait()

      def body0(r, accs):
        return tuple(a + rows_v[0, r, pl.ds(o, 16)] for a, o in zip(accs, _OFFS))

      def body1(r, accs):
        return tuple(a + rows_v[1, r, pl.ds(o, 16)] for a, o in zip(accs, _OFFS))

      zeros = tuple(jnp.zeros((16,), jnp.float32) for _ in _OFFS)
      accs = lax.fori_loop(0, _U0, body0, zeros)
      accs = lax.fori_loop(_U0 - _O1, _LL - 16, body1, accs)
      for a, o in zip(accs, _OFFS):
        pooled_v[e, pl.ds(o, 16)] = a * scale
      return carry

    lax.fori_loop(0, _BPW, elt_body, 0)
    pltpu.sync_copy(pooled_v, out_hbm.at[pl.ds(base, _BPW)])

  return k(idx_flat, table)


_BM = 512


def _mlp(pooled, W1, b1, W2, b2):
  def body(x_ref, w1_ref, b1_ref, w2_ref, b2_ref, o_ref):
    h = jnp.dot(x_ref[...], w1_ref[...], preferred_element_type=jnp.float32)
    h = h + b1_ref[...]
    o_ref[...] = (
        jnp.dot(h, w2_ref[...], preferred_element_type=jnp.float32) + b2_ref[...]
    )

  return pl.pallas_call(
      body,
      grid=(BATCH_N // _BM,),
      in_specs=[
          pl.BlockSpec((_BM, DIM_N), lambda i: (i, 0)),
          pl.BlockSpec((DIM_N, HIDDEN_N), lambda i: (0, 0)),
          pl.BlockSpec((1, HIDDEN_N), lambda i: (0, 0)),
          pl.BlockSpec((HIDDEN_N, CLASSES_N), lambda i: (0, 0)),
          pl.BlockSpec((1, CLASSES_N), lambda i: (0, 0)),
      ],
      out_specs=pl.BlockSpec((_BM, CLASSES_N), lambda i: (i, 0)),
      out_shape=jax.ShapeDtypeStruct((BATCH_N, CLASSES_N), jnp.float32),
  )(pooled, W1, b1, W2, b2)


def kernel(input_sentence, emb_table, W1, b1, W2, b2):
  idx_flat = jnp.concatenate(
      [input_sentence.reshape(-1), jnp.zeros((64,), input_sentence.dtype)])
  t104 = jnp.pad(emb_table, ((0, 0), (0, _DP - DIM_N)))
  pooled = _sc_mean_pool(idx_flat, t104)
  return _mlp(pooled, W1, b1.reshape(1, HIDDEN_N), W2, b2.reshape(1, CLASSES_N))


# speedup vs baseline: 1.6905x; 1.6905x over previous
import functools

import jax
import jax.numpy as jnp
from jax import lax
from jax.experimental import pallas as pl
from jax.experimental.pallas import tpu as pltpu
from jax.experimental.pallas import tpu_sc as plsc

VOCAB_N = 1000000
DIM_N = 100
HIDDEN_N = 150
CLASSES_N = 1000
BATCH_N = 4096
SEQ_N = 200

_NC = 2
_NS = 16
_NW = _NC * _NS
_BPW = BATCH_N // _NW
_OFFS = (0, 16, 32, 48, 64, 80, 84)
_LL = 128
_U0 = 112
_O1 = 88
_U1 = SEQ_N - _U0


_DP = 128


def _repack(table):
  t_t = table.T
  vb = 512

  def body(x_ref, o_ref):
    o_ref[:, 0:DIM_N] = x_ref[...].T

  return pl.pallas_call(
      body,
      grid=(pl.cdiv(VOCAB_N, vb),),
      in_specs=[pl.BlockSpec((DIM_N, vb), lambda j: (0, j))],
      out_specs=pl.BlockSpec((vb, _DP), lambda j: (j, 0)),
      out_shape=jax.ShapeDtypeStruct((VOCAB_N, _DP), jnp.float32),
  )(t_t)


def _sc_mean_pool(idx_flat, table):
  mesh = plsc.VectorSubcoreMesh(core_axis_name="c", subcore_axis_name="s")

  @functools.partial(
      pl.kernel,
      mesh=mesh,
      out_type=jax.ShapeDtypeStruct((BATCH_N, DIM_N), jnp.float32),
      scratch_types=[
          pltpu.VMEM((2, _LL), jnp.int32),
          pltpu.VMEM((2, _LL, _DP), jnp.float32),
          pltpu.VMEM((_BPW, DIM_N), jnp.float32),
          pltpu.SemaphoreType.DMA,
          pltpu.SemaphoreType.DMA,
      ],
  )
  def k(idx_hbm, table_hbm, out_hbm, idx_v, rows_v, pooled_v, sem0, sem1):
    wid = lax.axis_index("s") * _NC + lax.axis_index("c")
    base = wid * _BPW
    scale = jnp.float32(1.0 / SEQ_N)

    def elt_body(e, carry):
      fb = (base + e) * SEQ_N
      pltpu.sync_copy(idx_hbm.at[pl.ds(fb, _LL)], idx_v.at[0])
      pltpu.sync_copy(idx_hbm.at[pl.ds(fb + _O1, _LL)], idx_v.at[1])
      cp0 = pltpu.async_copy(table_hbm.at[idx_v.at[0]], rows_v.at[0], sem0)
      cp1 = pltpu.async_copy(table_hbm.at[idx_v.at[1]], rows_v.at[1], sem1)
      cp0.wait()
      cp1.wait()

      def body0(r, accs):
        return tuple(a + rows_v[0, r, pl.ds(o, 16)] for a, o in zip(accs, _OFFS))

      def body1(r, accs):
        return tuple(a + rows_v[1, r, pl.ds(o, 16)] for a, o in zip(accs, _OFFS))

      zeros = tuple(jnp.zeros((16,), jnp.float32) for _ in _OFFS)
      accs = lax.fori_loop(0, _U0, body0, zeros)
      accs = lax.fori_loop(_U0 - _O1, _LL - 16, body1, accs)
      for a, o in zip(accs, _OFFS):
        pooled_v[e, pl.ds(o, 16)] = a * scale
      return carry

    lax.fori_loop(0, _BPW, elt_body, 0)
    pltpu.sync_copy(pooled_v, out_hbm.at[pl.ds(base, _BPW)])

  return k(idx_flat, table)


_BM = 512


def _mlp(pooled, W1, b1, W2, b2):
  def body(x_ref, w1_ref, b1_ref, w2_ref, b2_ref, o_ref):
    h = jnp.dot(x_ref[...], w1_ref[...], preferred_element_type=jnp.float32)
    h = h + b1_ref[...]
    o_ref[...] = (
        jnp.dot(h, w2_ref[...], preferred_element_type=jnp.float32) + b2_ref[...]
    )

  return pl.pallas_call(
      body,
      grid=(BATCH_N // _BM,),
      in_specs=[
          pl.BlockSpec((_BM, DIM_N), lambda i: (i, 0)),
          pl.BlockSpec((DIM_N, HIDDEN_N), lambda i: (0, 0)),
          pl.BlockSpec((1, HIDDEN_N), lambda i: (0, 0)),
          pl.BlockSpec((HIDDEN_N, CLASSES_N), lambda i: (0, 0)),
          pl.BlockSpec((1, CLASSES_N), lambda i: (0, 0)),
      ],
      out_specs=pl.BlockSpec((_BM, CLASSES_N), lambda i: (i, 0)),
      out_shape=jax.ShapeDtypeStruct((BATCH_N, CLASSES_N), jnp.float32),
  )(pooled, W1, b1, W2, b2)


def kernel(input_sentence, emb_table, W1, b1, W2, b2):
  idx_flat = jnp.concatenate(
      [input_sentence.reshape(-1), jnp.zeros((64,), input_sentence.dtype)])
  pooled = _sc_mean_pool(idx_flat, _repack(emb_table))
  return _mlp(pooled, W1, b1.reshape(1, HIDDEN_N), W2, b2.reshape(1, CLASSES_N))


# speedup vs baseline: 4.5202x; 2.6738x over previous
import functools

import jax
import jax.numpy as jnp
from jax import lax
from jax.experimental import pallas as pl
from jax.experimental.pallas import tpu as pltpu
from jax.experimental.pallas import tpu_sc as plsc

VOCAB_N = 1000000
DIM_N = 100
HIDDEN_N = 150
CLASSES_N = 1000
BATCH_N = 4096
SEQ_N = 200

_NC = 2
_NS = 16
_NW = _NC * _NS
_BPW = BATCH_N // _NW
_OFFS = (0, 16, 32, 48, 64, 80, 84)
_LL = 128
_U0 = 112
_O1 = 88
_U1 = SEQ_N - _U0


_DP = 128


def _repack(table):
  t_t = table.T
  vb = 4096

  def body(x_ref, o_ref):
    o_ref[:, 0:DIM_N] = x_ref[...].T

  return pl.pallas_call(
      body,
      grid=(pl.cdiv(VOCAB_N, vb),),
      in_specs=[pl.BlockSpec((DIM_N, vb), lambda j: (0, j))],
      out_specs=pl.BlockSpec((vb, _DP), lambda j: (j, 0)),
      out_shape=jax.ShapeDtypeStruct((VOCAB_N, _DP), jnp.float32),
  )(t_t)


def _sc_mean_pool(idx_flat, table):
  mesh = plsc.VectorSubcoreMesh(core_axis_name="c", subcore_axis_name="s")

  @functools.partial(
      pl.kernel,
      mesh=mesh,
      out_type=jax.ShapeDtypeStruct((BATCH_N, DIM_N), jnp.float32),
      scratch_types=[
          pltpu.VMEM((2, 2, _LL), jnp.int32),
          pltpu.VMEM((2, 2, _LL, _DP), jnp.float32),
          pltpu.VMEM((_BPW, DIM_N), jnp.float32),
          pltpu.SemaphoreType.DMA,
          pltpu.SemaphoreType.DMA,
          pltpu.SemaphoreType.DMA,
          pltpu.SemaphoreType.DMA,
      ],
  )
  def k(idx_hbm, table_hbm, out_hbm, idx_v, rows_v, pooled_v,
        isem0, isem1, gsem0, gsem1):
    wid = lax.axis_index("s") * _NC + lax.axis_index("c")
    base = wid * _BPW
    scale = jnp.float32(1.0 / SEQ_N)
    isems = (isem0, isem1)
    gsems = (gsem0, gsem1)

    def idx_copies(e, slot):
      fb = (base + e) * SEQ_N
      return (
          (idx_hbm.at[pl.ds(fb, _LL)], idx_v.at[slot, 0], isems[slot]),
          (idx_hbm.at[pl.ds(fb + _O1, _LL)], idx_v.at[slot, 1], isems[slot]),
      )

    def stage(e, slot):
      for src, dst, sem in idx_copies(e, slot):
        pltpu.async_copy(src, dst, sem)

    def wait_stage(e, slot):
      for src, dst, sem in idx_copies(e, slot):
        pltpu.make_async_copy(src, dst, sem).wait()

    def gather_copies(slot):
      return (
          (table_hbm.at[idx_v.at[slot, 0]], rows_v.at[slot, 0], gsems[slot]),
          (table_hbm.at[idx_v.at[slot, 1]], rows_v.at[slot, 1], gsems[slot]),
      )

    def fire(slot):
      for src, dst, sem in gather_copies(slot):
        pltpu.async_copy(src, dst, sem)

    def wait_gather(slot):
      for src, dst, sem in gather_copies(slot):
        pltpu.make_async_copy(src, dst, sem).wait()

    def accumulate(e, slot):
      def body0(r, accs):
        return tuple(
            a + rows_v[slot, 0, r, pl.ds(o, 16)] for a, o in zip(accs, _OFFS))

      def body1(r, accs):
        return tuple(
            a + rows_v[slot, 1, r, pl.ds(o, 16)] for a, o in zip(accs, _OFFS))

      zeros = tuple(jnp.zeros((16,), jnp.float32) for _ in _OFFS)
      accs = lax.fori_loop(0, _U0, body0, zeros)
      accs = lax.fori_loop(_U0 - _O1, _LL - 16, body1, accs)
      for a, o in zip(accs, _OFFS):
        pooled_v[e, pl.ds(o, 16)] = a * scale

    stage(0, 0)
    wait_stage(0, 0)
    fire(0)
    stage(1, 1)

    def pair_body(g, carry):
      e0 = 2 * g
      e1 = e0 + 1
      wait_stage(e1, 1)
      fire(1)
      wait_gather(0)

      @pl.when(g < _BPW // 2 - 1)
      def _():
        stage(e0 + 2, 0)

      accumulate(e0, 0)

      @pl.when(g < _BPW // 2 - 1)
      def _():
        wait_stage(e0 + 2, 0)
        fire(0)
        stage(e1 + 2, 1)

      accumulate(e1, 1)
      return carry

    lax.fori_loop(0, _BPW // 2, pair_body, 0)
    pltpu.sync_copy(pooled_v, out_hbm.at[pl.ds(base, _BPW)])

  return k(idx_flat, table)


_BM = 512


def _mlp(pooled, W1, b1, W2, b2):
  def body(x_ref, w1_ref, b1_ref, w2_ref, b2_ref, o_ref):
    h = jnp.dot(x_ref[...], w1_ref[...], preferred_element_type=jnp.float32)
    h = h + b1_ref[...]
    o_ref[...] = (
        jnp.dot(h, w2_ref[...], preferred_element_type=jnp.float32) + b2_ref[...]
    )

  return pl.pallas_call(
      body,
      grid=(BATCH_N // _BM,),
      in_specs=[
          pl.BlockSpec((_BM, DIM_N), lambda i: (i, 0)),
          pl.BlockSpec((DIM_N, HIDDEN_N), lambda i: (0, 0)),
          pl.BlockSpec((1, HIDDEN_N), lambda i: (0, 0)),
          pl.BlockSpec((HIDDEN_N, CLASSES_N), lambda i: (0, 0)),
          pl.BlockSpec((1, CLASSES_N), lambda i: (0, 0)),
      ],
      out_specs=pl.BlockSpec((_BM, CLASSES_N), lambda i: (i, 0)),
      out_shape=jax.ShapeDtypeStruct((BATCH_N, CLASSES_N), jnp.float32),
  )(pooled, W1, b1, W2, b2)


def kernel(input_sentence, emb_table, W1, b1, W2, b2):
  idx_flat = jnp.concatenate(
      [input_sentence.reshape(-1), jnp.zeros((64,), input_sentence.dtype)])
  pooled = _sc_mean_pool(idx_flat, _repack(emb_table))
  return _mlp(pooled, W1, b1.reshape(1, HIDDEN_N), W2, b2.reshape(1, CLASSES_N))


# speedup vs baseline: 4.5867x; 1.0147x over previous
import functools

import jax
import jax.numpy as jnp
from jax import lax
from jax.experimental import pallas as pl
from jax.experimental.pallas import tpu as pltpu
from jax.experimental.pallas import tpu_sc as plsc

VOCAB_N = 1000000
DIM_N = 100
HIDDEN_N = 150
CLASSES_N = 1000
BATCH_N = 4096
SEQ_N = 200

_NC = 2
_NS = 16
_NW = _NC * _NS
_BPW = BATCH_N // _NW
_OFFS = (0, 16, 32, 48, 64, 80, 84)
_LL = 128
_U0 = 112
_O1 = 88
_U1 = SEQ_N - _U0


_DP = 128


def _repack(table):
  t_t = table.T
  vb = 4096

  def body(x_ref, o_ref):
    o_ref[:, 0:DIM_N] = x_ref[...].T

  return pl.pallas_call(
      body,
      grid=(pl.cdiv(VOCAB_N, vb),),
      in_specs=[pl.BlockSpec((DIM_N, vb), lambda j: (0, j))],
      out_specs=pl.BlockSpec((vb, _DP), lambda j: (j, 0)),
      out_shape=jax.ShapeDtypeStruct((VOCAB_N, _DP), jnp.float32),
  )(t_t)


def _sc_mean_pool(idx_flat, table):
  mesh = plsc.VectorSubcoreMesh(core_axis_name="c", subcore_axis_name="s")

  @functools.partial(
      pl.kernel,
      mesh=mesh,
      out_type=jax.ShapeDtypeStruct((BATCH_N, DIM_N), jnp.float32),
      scratch_types=[
          pltpu.VMEM((2, 2, _LL), jnp.int32),
          pltpu.VMEM((2, 2, _LL, _DP), jnp.float32),
          pltpu.VMEM((_BPW, DIM_N), jnp.float32),
          pltpu.SemaphoreType.DMA,
          pltpu.SemaphoreType.DMA,
          pltpu.SemaphoreType.DMA,
          pltpu.SemaphoreType.DMA,
      ],
  )
  def k(idx_hbm, table_hbm, out_hbm, idx_v, rows_v, pooled_v,
        isem0, isem1, gsem0, gsem1):
    wid = lax.axis_index("s") * _NC + lax.axis_index("c")
    base = wid * _BPW
    scale = jnp.float32(1.0 / SEQ_N)
    isems = (isem0, isem1)
    gsems = (gsem0, gsem1)

    def idx_copies(e, slot):
      fb = (base + e) * SEQ_N
      return (
          (idx_hbm.at[pl.ds(fb, _LL)], idx_v.at[slot, 0], isems[slot]),
          (idx_hbm.at[pl.ds(fb + _O1, _LL)], idx_v.at[slot, 1], isems[slot]),
      )

    def stage(e, slot):
      for src, dst, sem in idx_copies(e, slot):
        pltpu.async_copy(src, dst, sem)

    def wait_stage(e, slot):
      for src, dst, sem in idx_copies(e, slot):
        pltpu.make_async_copy(src, dst, sem).wait()

    def gather_copies(slot):
      return (
          (table_hbm.at[idx_v.at[slot, 0]], rows_v.at[slot, 0], gsems[slot]),
          (table_hbm.at[idx_v.at[slot, 1]], rows_v.at[slot, 1], gsems[slot]),
      )

    def fire(slot):
      for src, dst, sem in gather_copies(slot):
        pltpu.async_copy(src, dst, sem)

    def wait_gather(slot):
      for src, dst, sem in gather_copies(slot):
        pltpu.make_async_copy(src, dst, sem).wait()

    def accumulate(e, slot):
      def body0(r, accs):
        return tuple(
            a + rows_v[slot, 0, r, pl.ds(o, 16)] for a, o in zip(accs, _OFFS))

      def body1(r, accs):
        return tuple(
            a + rows_v[slot, 1, r, pl.ds(o, 16)] for a, o in zip(accs, _OFFS))

      zeros = tuple(jnp.zeros((16,), jnp.float32) for _ in _OFFS)
      accs = lax.fori_loop(0, _U0, body0, zeros)
      accs = lax.fori_loop(_U0 - _O1, _LL - 16, body1, accs)
      for a, o in zip(accs, _OFFS):
        pooled_v[e, pl.ds(o, 16)] = a * scale

    stage(0, 0)
    wait_stage(0, 0)
    fire(0)
    stage(1, 1)

    def pair_body(g, carry):
      e0 = 2 * g
      e1 = e0 + 1
      wait_stage(e1, 1)
      fire(1)
      wait_gather(0)

      @pl.when(g < _BPW // 2 - 1)
      def _():
        stage(e0 + 2, 0)

      accumulate(e0, 0)

      @pl.when(g < _BPW // 2 - 1)
      def _():
        wait_stage(e0 + 2, 0)
        fire(0)

      wait_gather(1)

      @pl.when(g < _BPW // 2 - 1)
      def _():
        stage(e1 + 2, 1)

      accumulate(e1, 1)
      return carry

    lax.fori_loop(0, _BPW // 2, pair_body, 0)
    pltpu.sync_copy(pooled_v, out_hbm.at[pl.ds(base, _BPW)])

  return k(idx_flat, table)


_BM = 512


def _mlp(pooled, W1, b1, W2, b2):
  def body(x_ref, w1_ref, b1_ref, w2_ref, b2_ref, o_ref):
    h = jnp.dot(x_ref[...], w1_ref[...], preferred_element_type=jnp.float32)
    h = h + b1_ref[...]
    o_ref[...] = (
        jnp.dot(h, w2_ref[...], preferred_element_type=jnp.float32) + b2_ref[...]
    )

  return pl.pallas_call(
      body,
      grid=(BATCH_N // _BM,),
      in_specs=[
          pl.BlockSpec((_BM, DIM_N), lambda i: (i, 0)),
          pl.BlockSpec((DIM_N, HIDDEN_N), lambda i: (0, 0)),
          pl.BlockSpec((1, HIDDEN_N), lambda i: (0, 0)),
          pl.BlockSpec((HIDDEN_N, CLASSES_N), lambda i: (0, 0)),
          pl.BlockSpec((1, CLASSES_N), lambda i: (0, 0)),
      ],
      out_specs=pl.BlockSpec((_BM, CLASSES_N), lambda i: (i, 0)),
      out_shape=jax.ShapeDtypeStruct((BATCH_N, CLASSES_N), jnp.float32),
  )(pooled, W1, b1, W2, b2)


def kernel(input_sentence, emb_table, W1, b1, W2, b2):
  idx_flat = jnp.concatenate(
      [input_sentence.reshape(-1), jnp.zeros((64,), input_sentence.dtype)])
  pooled = _sc_mean_pool(idx_flat, _repack(emb_table))
  return _mlp(pooled, W1, b1.reshape(1, HIDDEN_N), W2, b2.reshape(1, CLASSES_N))


# speedup vs baseline: 5.5440x; 1.2087x over previous
import functools

import jax
import jax.numpy as jnp
from jax import lax
from jax.experimental import pallas as pl
from jax.experimental.pallas import tpu as pltpu
from jax.experimental.pallas import tpu_sc as plsc

VOCAB_N = 1000000
DIM_N = 100
HIDDEN_N = 150
CLASSES_N = 1000
BATCH_N = 4096
SEQ_N = 200

_NC = 2
_NS = 16
_NW = _NC * _NS
_BPW = BATCH_N // _NW
_OFFS = (0, 16, 32, 48, 64, 80, 84)
_LL = 128
_U0 = 112
_O1 = 88
_U1 = SEQ_N - _U0


_DP = 128


def _repack(table):
  t_t = table.T
  vb = 8192

  def body(x_ref, o_ref):
    o_ref[:, 0:DIM_N] = x_ref[...].T

  return pl.pallas_call(
      body,
      grid=(pl.cdiv(VOCAB_N, vb),),
      in_specs=[pl.BlockSpec((DIM_N, vb), lambda j: (0, j))],
      out_specs=pl.BlockSpec((vb, _DP), lambda j: (j, 0)),
      out_shape=jax.ShapeDtypeStruct((VOCAB_N, _DP), jnp.float32),
  )(t_t)


def _sc_mean_pool(idx_flat, table):
  mesh = plsc.VectorSubcoreMesh(core_axis_name="c", subcore_axis_name="s")

  @functools.partial(
      pl.kernel,
      mesh=mesh,
      out_type=jax.ShapeDtypeStruct((BATCH_N, DIM_N), jnp.float32),
      scratch_types=[
          pltpu.VMEM((2, 2, _LL), jnp.int32),
          pltpu.VMEM((2, 2, _LL, _DP), jnp.float32),
          pltpu.VMEM((_BPW, DIM_N), jnp.float32),
          pltpu.SemaphoreType.DMA,
          pltpu.SemaphoreType.DMA,
          pltpu.SemaphoreType.DMA,
          pltpu.SemaphoreType.DMA,
      ],
  )
  def k(idx_hbm, table_hbm, out_hbm, idx_v, rows_v, pooled_v,
        isem0, isem1, gsem0, gsem1):
    wid = lax.axis_index("s") * _NC + lax.axis_index("c")
    base = wid * _BPW
    scale = jnp.float32(1.0 / SEQ_N)
    isems = (isem0, isem1)
    gsems = (gsem0, gsem1)

    def idx_copies(e, slot):
      fb = (base + e) * SEQ_N
      return (
          (idx_hbm.at[pl.ds(fb, _LL)], idx_v.at[slot, 0], isems[slot]),
          (idx_hbm.at[pl.ds(fb + _O1, _LL)], idx_v.at[slot, 1], isems[slot]),
      )

    def stage(e, slot):
      for src, dst, sem in idx_copies(e, slot):
        pltpu.async_copy(src, dst, sem)

    def wait_stage(e, slot):
      for src, dst, sem in idx_copies(e, slot):
        pltpu.make_async_copy(src, dst, sem).wait()

    def gather_copies(slot):
      return (
          (table_hbm.at[idx_v.at[slot, 0]], rows_v.at[slot, 0], gsems[slot]),
          (table_hbm.at[idx_v.at[slot, 1]], rows_v.at[slot, 1], gsems[slot]),
      )

    def fire(slot):
      for src, dst, sem in gather_copies(slot):
        pltpu.async_copy(src, dst, sem)

    def wait_gather(slot):
      for src, dst, sem in gather_copies(slot):
        pltpu.make_async_copy(src, dst, sem).wait()

    def accumulate(e, slot):
      def body0(r, accs):
        return tuple(
            a + rows_v[slot, 0, r, pl.ds(o, 16)] for a, o in zip(accs, _OFFS))

      def body1(r, accs):
        return tuple(
            a + rows_v[slot, 1, r, pl.ds(o, 16)] for a, o in zip(accs, _OFFS))

      zeros = tuple(jnp.zeros((16,), jnp.float32) for _ in _OFFS)
      accs = lax.fori_loop(0, _U0, body0, zeros)
      accs = lax.fori_loop(_U0 - _O1, _LL - 16, body1, accs)
      for a, o in zip(accs, _OFFS):
        pooled_v[e, pl.ds(o, 16)] = a * scale

    stage(0, 0)
    wait_stage(0, 0)
    fire(0)
    stage(1, 1)

    def pair_body(g, carry):
      e0 = 2 * g
      e1 = e0 + 1
      wait_stage(e1, 1)
      fire(1)
      wait_gather(0)

      @pl.when(g < _BPW // 2 - 1)
      def _():
        stage(e0 + 2, 0)

      accumulate(e0, 0)

      @pl.when(g < _BPW // 2 - 1)
      def _():
        wait_stage(e0 + 2, 0)
        fire(0)

      wait_gather(1)

      @pl.when(g < _BPW // 2 - 1)
      def _():
        stage(e1 + 2, 1)

      accumulate(e1, 1)
      return carry

    lax.fori_loop(0, _BPW // 2, pair_body, 0)
    pltpu.sync_copy(pooled_v, out_hbm.at[pl.ds(base, _BPW)])

  return k(idx_flat, table)


_BM = 512


def _mlp(pooled, W1, b1, W2, b2):
  def body(x_ref, w1_ref, b1_ref, w2_ref, b2_ref, o_ref):
    h = jnp.dot(x_ref[...], w1_ref[...], preferred_element_type=jnp.float32)
    h = h + b1_ref[...]
    o_ref[...] = (
        jnp.dot(h, w2_ref[...], preferred_element_type=jnp.float32) + b2_ref[...]
    )

  return pl.pallas_call(
      body,
      grid=(BATCH_N // _BM,),
      in_specs=[
          pl.BlockSpec((_BM, DIM_N), lambda i: (i, 0)),
          pl.BlockSpec((DIM_N, HIDDEN_N), lambda i: (0, 0)),
          pl.BlockSpec((1, HIDDEN_N), lambda i: (0, 0)),
          pl.BlockSpec((HIDDEN_N, CLASSES_N), lambda i: (0, 0)),
          pl.BlockSpec((1, CLASSES_N), lambda i: (0, 0)),
      ],
      out_specs=pl.BlockSpec((_BM, CLASSES_N), lambda i: (i, 0)),
      out_shape=jax.ShapeDtypeStruct((BATCH_N, CLASSES_N), jnp.float32),
  )(pooled, W1, b1, W2, b2)


def kernel(input_sentence, emb_table, W1, b1, W2, b2):
  idx_flat = jnp.concatenate(
      [input_sentence.reshape(-1), jnp.zeros((64,), input_sentence.dtype)])
  pooled = _sc_mean_pool(idx_flat, _repack(emb_table))
  return _mlp(pooled, W1, b1.reshape(1, HIDDEN_N), W2, b2.reshape(1, CLASSES_N))
